# Initial kernel scaffold; baseline (speedup 1.0000x reference)
import jax
import jax.numpy as jnp
from jax.experimental import pallas as pl


def kernel(padded, lengths):
    raise NotImplementedError("write your pallas kernel here")


# speedup vs baseline: 3.6979x; 3.6979x over previous
import jax
import jax.numpy as jnp
from jax import lax
from jax.experimental import pallas as pl
from jax.experimental.pallas import tpu as pltpu
from jax.experimental.pallas import tpu_sc as plsc

_RATE = 2
_B, _T, _D = 16, 4096, 256
_TO = _T // _RATE
_C = 128
_NWORK = 32
_RPW = (_B * _TO) // _NWORK
_NCH = _RPW // _C
_L = 16


def _sc_body(padded_hbm, lengths_hbm, out_hbm, nl_hbm,
             idx_v, gbuf, zbuf, lens_v, nl_v, sem):
    wid = lax.axis_index("s") * 2 + lax.axis_index("c")
    b = wid // 2
    h = wid % 2
    base = h * _RPW
    grow0 = b * _TO + base

    pltpu.sync_copy(lengths_hbm, lens_v)
    lane = lax.iota(jnp.int32, _L)
    myl = jnp.sum(jnp.where(lane == b, lens_v[...], 0))
    nl = myl // _RATE
    v = jnp.clip(nl - base, 0, _RPW)

    @pl.when(wid == 0)
    def _():
        nl_v[...] = lens_v[...] // _RATE
        pltpu.sync_copy(nl_v, nl_hbm)

    zeros16 = jnp.zeros((_L,), jnp.float32)

    @pl.loop(0, _C)
    def _(r):
        for j in range(_D // _L):
            zbuf[r, pl.ds(j * _L, _L)] = zeros16

    for k in range(_NCH):
        vk = jnp.clip(v - k * _C, 0, _C)
        orow = grow0 + k * _C

        @pl.when(vk > 0)
        def _(vk=vk, k=k, orow=orow):
            row0 = b * _T + 2 * (base + k * _C) + 1
            for j in range(_C // _L):
                idx_v[pl.ds(j * _L, _L)] = row0 + 2 * (j * _L + lane)
            pltpu.async_copy(padded_hbm.at[idx_v], gbuf, sem).wait()

            @pl.when(vk < _C)
            def _():
                @pl.loop(vk, _C)
                def _(r):
                    for j in range(_D // _L):
                        gbuf[r, pl.ds(j * _L, _L)] = zeros16

            pltpu.sync_copy(gbuf, out_hbm.at[pl.ds(orow, _C)])

        @pl.when(vk == 0)
        def _(orow=orow):
            pltpu.sync_copy(zbuf, out_hbm.at[pl.ds(orow, _C)])


def kernel(padded, lengths):
    padded2d = padded.reshape(_B * _T, _D)
    mesh = plsc.VectorSubcoreMesh(core_axis_name="c", subcore_axis_name="s")
    out2d, nl = pl.kernel(
        _sc_body,
        out_type=(
            jax.ShapeDtypeStruct((_B * _TO, _D), jnp.float32),
            jax.ShapeDtypeStruct((_B,), jnp.int32),
        ),
        mesh=mesh,
        compiler_params=pltpu.CompilerParams(needs_layout_passes=False),
        scratch_types=(
            pltpu.VMEM((_C,), jnp.int32),
            pltpu.VMEM((_C, _D), jnp.float32),
            pltpu.VMEM((_C, _D), jnp.float32),
            pltpu.VMEM((_L,), jnp.int32),
            pltpu.VMEM((_L,), jnp.int32),
            pltpu.SemaphoreType.DMA,
        ),
    )(padded2d, lengths)
    return out2d.reshape(_B, _TO, _D), nl


# speedup vs baseline: 4.3259x; 1.1698x over previous
import jax
import jax.numpy as jnp
from jax import lax
from jax.experimental import pallas as pl
from jax.experimental.pallas import tpu as pltpu
from jax.experimental.pallas import tpu_sc as plsc

_RATE = 2
_B, _T, _D = 16, 4096, 256
_TO = _T // _RATE
_C = 128
_NWORK = 32
_RPW = (_B * _TO) // _NWORK
_NCH = _RPW // _C
_L = 16


def _sc_body(padded_hbm, lengths_hbm, out_hbm, nl_hbm,
             idx0, idx1, gbuf0, gbuf1, zbuf, lens_v, nl_v,
             gsem0, gsem1, osem0, osem1, zsem):
    idxb = (idx0, idx1)
    gbuf = (gbuf0, gbuf1)
    gsem = (gsem0, gsem1)
    osem = (osem0, osem1)

    wid = lax.axis_index("s") * 2 + lax.axis_index("c")
    b = wid // 2
    h = wid % 2
    base = h * _RPW
    grow0 = b * _TO + base

    pltpu.sync_copy(lengths_hbm, lens_v)
    lane = lax.iota(jnp.int32, _L)
    myl = jnp.sum(jnp.where(lane == b, lens_v[...], 0))
    nl = myl // _RATE
    v = jnp.clip(nl - base, 0, _RPW)

    @pl.when(wid == 0)
    def _():
        nl_v[...] = lens_v[...] // _RATE
        pltpu.sync_copy(nl_v, nl_hbm)

    zeros16 = jnp.zeros((_L,), jnp.float32)

    @pl.loop(0, _C)
    def _(r):
        for j in range(_D // _L):
            zbuf[r, pl.ds(j * _L, _L)] = zeros16

    vk = [jnp.clip(v - k * _C, 0, _C) for k in range(_NCH)]
    orow = [grow0 + k * _C for k in range(_NCH)]

    for k in range(_NCH):
        @pl.when(vk[k] == 0)
        def _(k=k):
            pltpu.async_copy(zbuf, out_hbm.at[pl.ds(orow[k], _C)], zsem)

    def start_gather(k):
        s = k % 2
        row0 = b * _T + 2 * (base + k * _C) + 1
        for j in range(_C // _L):
            idxb[s][pl.ds(j * _L, _L)] = row0 + 2 * (j * _L + lane)
        pltpu.async_copy(padded_hbm.at[idxb[s]], gbuf[s], gsem[s])

    def finish_gather_start_out(k):
        s = k % 2
        pltpu.make_async_copy(padded_hbm.at[idxb[s]], gbuf[s], gsem[s]).wait()

        @pl.when(vk[k] < _C)
        def _():
            @pl.loop(vk[k], _C)
            def _(r):
                for j in range(_D // _L):
                    gbuf[s][r, pl.ds(j * _L, _L)] = zeros16

        pltpu.async_copy(gbuf[s], out_hbm.at[pl.ds(orow[k], _C)], osem[s])

    def finish_out(k):
        s = k % 2
        pltpu.make_async_copy(
            gbuf[s], out_hbm.at[pl.ds(orow[k], _C)], osem[s]).wait()

    for k in range(_NCH):
        @pl.when(vk[k] > 0)
        def _(k=k):
            if k >= 2:
                @pl.when(vk[k - 2] > 0)
                def _():
                    finish_out(k - 2)
            start_gather(k)
        if k >= 1:
            @pl.when(vk[k - 1] > 0)
            def _(k=k):
                finish_gather_start_out(k - 1)

    @pl.when(vk[_NCH - 1] > 0)
    def _():
        finish_gather_start_out(_NCH - 1)

    for k in (_NCH - 2, _NCH - 1):
        @pl.when(vk[k] > 0)
        def _(k=k):
            finish_out(k)

    for k in range(_NCH):
        @pl.when(vk[k] == 0)
        def _(k=k):
            pltpu.make_async_copy(
                zbuf, out_hbm.at[pl.ds(orow[k], _C)], zsem).wait()


def kernel(padded, lengths):
    padded2d = padded.reshape(_B * _T, _D)
    mesh = plsc.VectorSubcoreMesh(core_axis_name="c", subcore_axis_name="s")
    out2d, nl = pl.kernel(
        _sc_body,
        out_type=(
            jax.ShapeDtypeStruct((_B * _TO, _D), jnp.float32),
            jax.ShapeDtypeStruct((_B,), jnp.int32),
        ),
        mesh=mesh,
        compiler_params=pltpu.CompilerParams(needs_layout_passes=False),
        scratch_types=(
            pltpu.VMEM((_C,), jnp.int32),
            pltpu.VMEM((_C,), jnp.int32),
            pltpu.VMEM((_C, _D), jnp.float32),
            pltpu.VMEM((_C, _D), jnp.float32),
            pltpu.VMEM((_C, _D), jnp.float32),
            pltpu.VMEM((_L,), jnp.int32),
            pltpu.VMEM((_L,), jnp.int32),
            pltpu.SemaphoreType.DMA,
            pltpu.SemaphoreType.DMA,
            pltpu.SemaphoreType.DMA,
            pltpu.SemaphoreType.DMA,
            pltpu.SemaphoreType.DMA,
        ),
    )(padded2d, lengths)
    return out2d.reshape(_B, _TO, _D), nl
